# initial kernel scaffold (unmeasured)
import jax
import jax.numpy as jnp
from jax import lax
from jax.experimental import pallas as pl
from jax.experimental.pallas import tpu as pltpu

T_LOC = 512
D = 1024
F = 2048
E_LOC = 4
NY = 2
F_BLK = 512
NF = F // F_BLK


def _exchange(x, router):

    def body(x_ref, r_ref, xall_ref, rall_ref, send_sems, recv_sems):
        my_x = lax.axis_index("x")
        my_y = lax.axis_index("y")
        peer = (my_x, 1 - my_y)

        barrier = pltpu.get_barrier_semaphore()
        pl.semaphore_signal(barrier, inc=1, device_id=peer,
                            device_id_type=pl.DeviceIdType.MESH)
        pl.semaphore_wait(barrier, 1)

        xall_ref[my_y] = x_ref[...]
        rall_ref[my_y] = r_ref[...]

        rdma_x = pltpu.make_async_remote_copy(
            src_ref=x_ref,
            dst_ref=xall_ref.at[my_y],
            send_sem=send_sems.at[0],
            recv_sem=recv_sems.at[0],
            device_id=peer,
            device_id_type=pl.DeviceIdType.MESH,
        )
        rdma_x.start()
        rdma_r = pltpu.make_async_remote_copy(
            src_ref=r_ref,
            dst_ref=rall_ref.at[my_y],
            send_sem=send_sems.at[1],
            recv_sem=recv_sems.at[1],
            device_id=peer,
            device_id_type=pl.DeviceIdType.MESH,
        )
        rdma_r.start()
        rdma_x.wait()
        rdma_r.wait()

    return pl.pallas_call(
        body,
        out_shape=(
            jax.ShapeDtypeStruct((NY, T_LOC, D), jnp.float32),
            jax.ShapeDtypeStruct((NY, D, E_LOC), jnp.float32),
        ),
        in_specs=[
            pl.BlockSpec(memory_space=pltpu.VMEM),
            pl.BlockSpec(memory_space=pltpu.VMEM),
        ],
        out_specs=(
            pl.BlockSpec(memory_space=pltpu.VMEM),
            pl.BlockSpec(memory_space=pltpu.VMEM),
        ),
        scratch_shapes=[
            pltpu.SemaphoreType.DMA((2,)),
            pltpu.SemaphoreType.DMA((2,)),
        ],
        compiler_params=pltpu.CompilerParams(collective_id=0),
    )(x, router)


def _ffn(xall, wloc, W1, W2):

    def body(x_ref, w_ref, w1_ref, w2_ref, out_ref):
        e = pl.program_id(0)
        f = pl.program_id(1)

        @pl.when(jnp.logical_and(e == 0, f == 0))
        def _():
            out_ref[...] = jnp.zeros_like(out_ref)

        h = jnp.maximum(
            jnp.dot(x_ref[...], w1_ref[0], preferred_element_type=jnp.float32),
            0.0,
        )
        y = jnp.dot(h, w2_ref[0], preferred_element_type=jnp.float32)
        out_ref[...] += y * w_ref[...]

    return pl.pallas_call(
        body,
        grid=(E_LOC, NF),
        in_specs=[
            pl.BlockSpec((NY * T_LOC, D), lambda e, f: (0, 0)),
            pl.BlockSpec((NY * T_LOC, 1), lambda e, f: (0, e)),
            pl.BlockSpec((1, D, F_BLK), lambda e, f: (e, 0, f)),
            pl.BlockSpec((1, F_BLK, D), lambda e, f: (e, f, 0)),
        ],
        out_specs=pl.BlockSpec((NY * T_LOC, D), lambda e, f: (0, 0)),
        out_shape=jax.ShapeDtypeStruct((NY * T_LOC, D), jnp.float32),
    )(xall, wloc, W1, W2)


def _combine(partial):

    def body(p_ref, out_ref, comm_ref, send_sem, recv_sem):
        my_x = lax.axis_index("x")
        my_y = lax.axis_index("y")
        peer = (my_x, 1 - my_y)

        barrier = pltpu.get_barrier_semaphore()
        pl.semaphore_signal(barrier, inc=1, device_id=peer,
                            device_id_type=pl.DeviceIdType.MESH)
        pl.semaphore_wait(barrier, 1)

        rdma = pltpu.make_async_remote_copy(
            src_ref=p_ref.at[1 - my_y],
            dst_ref=comm_ref,
            send_sem=send_sem,
            recv_sem=recv_sem,
            device_id=peer,
            device_id_type=pl.DeviceIdType.MESH,
        )
        rdma.start()
        rdma.wait()
        out_ref[...] = p_ref[my_y] + comm_ref[...]

    return pl.pallas_call(
        body,
        out_shape=jax.ShapeDtypeStruct((T_LOC, D), jnp.float32),
        in_specs=[pl.BlockSpec(memory_space=pltpu.VMEM)],
        out_specs=pl.BlockSpec(memory_space=pltpu.VMEM),
        scratch_shapes=[
            pltpu.VMEM((T_LOC, D), jnp.float32),
            pltpu.SemaphoreType.DMA,
            pltpu.SemaphoreType.DMA,
        ],
        compiler_params=pltpu.CompilerParams(collective_id=1),
    )(partial)


def kernel(x, router, W1, W2):
    xall_slots, rall_slots = _exchange(x, router)
    xall = xall_slots.reshape(NY * T_LOC, D)
    rfull = jnp.concatenate([rall_slots[0], rall_slots[1]], axis=1)

    gates = xall @ rfull
    top_v, top_i = lax.top_k(gates, 2)
    w = jnp.exp(top_v - top_v.max(axis=1, keepdims=True))
    w = w / w.sum(axis=1, keepdims=True)

    my_y = lax.axis_index("y")
    eids = my_y * E_LOC + jnp.arange(E_LOC)
    wloc = jnp.sum(
        w[:, :, None] * (top_i[:, :, None] == eids[None, None, :]), axis=1
    )

    partial = _ffn(xall, wloc, W1, W2)
    return _combine(partial.reshape(NY, T_LOC, D))


# baseline (device time: 109598 ns/iter reference)
import jax
import jax.numpy as jnp
from jax import lax
from jax.experimental import pallas as pl
from jax.experimental.pallas import tpu as pltpu

T_LOC = 512
D = 1024
F = 2048
E_LOC = 4
NY = 2
F_BLK = 512
NF = F // F_BLK


def _exchange(x, router):

    def body(x_ref, r_ref, xall_ref, rall_ref, send_sems, recv_sems):
        my_x = lax.axis_index("x")
        my_y = lax.axis_index("y")
        peer = (my_x, 1 - my_y)

        barrier = pltpu.get_barrier_semaphore()
        pl.semaphore_signal(barrier, inc=1, device_id=peer,
                            device_id_type=pl.DeviceIdType.MESH)
        pl.semaphore_wait(barrier, 1)

        xall_ref[my_y] = x_ref[...]
        rall_ref[my_y] = r_ref[...]

        rdma_x = pltpu.make_async_remote_copy(
            src_ref=x_ref,
            dst_ref=xall_ref.at[my_y],
            send_sem=send_sems.at[0],
            recv_sem=recv_sems.at[0],
            device_id=peer,
            device_id_type=pl.DeviceIdType.MESH,
        )
        rdma_x.start()
        rdma_r = pltpu.make_async_remote_copy(
            src_ref=r_ref,
            dst_ref=rall_ref.at[my_y],
            send_sem=send_sems.at[1],
            recv_sem=recv_sems.at[1],
            device_id=peer,
            device_id_type=pl.DeviceIdType.MESH,
        )
        rdma_r.start()
        rdma_x.wait()
        rdma_r.wait()

    return pl.pallas_call(
        body,
        out_shape=(
            jax.ShapeDtypeStruct((NY, T_LOC, D), jnp.float32),
            jax.ShapeDtypeStruct((NY, D, E_LOC), jnp.float32),
        ),
        in_specs=[
            pl.BlockSpec(memory_space=pltpu.VMEM),
            pl.BlockSpec(memory_space=pltpu.VMEM),
        ],
        out_specs=(
            pl.BlockSpec(memory_space=pltpu.VMEM),
            pl.BlockSpec(memory_space=pltpu.VMEM),
        ),
        scratch_shapes=[
            pltpu.SemaphoreType.DMA((2,)),
            pltpu.SemaphoreType.DMA((2,)),
        ],
        compiler_params=pltpu.CompilerParams(collective_id=0),
    )(x, router)


def _ffn(xall, wloc, W1, W2):

    def body(x_ref, w_ref, w1_ref, w2_ref, out_ref):
        e = pl.program_id(0)
        f = pl.program_id(1)

        @pl.when(jnp.logical_and(e == 0, f == 0))
        def _():
            out_ref[...] = jnp.zeros_like(out_ref)

        h = jnp.maximum(
            jnp.dot(x_ref[...], w1_ref[0], preferred_element_type=jnp.float32),
            0.0,
        )
        y = jnp.dot(h, w2_ref[0], preferred_element_type=jnp.float32)
        col = lax.broadcasted_iota(jnp.int32, (1, E_LOC), 1) == e
        wcol = jnp.sum(w_ref[...] * col.astype(jnp.float32), axis=1,
                       keepdims=True)
        out_ref[...] += y * wcol

    return pl.pallas_call(
        body,
        grid=(E_LOC, NF),
        in_specs=[
            pl.BlockSpec((NY * T_LOC, D), lambda e, f: (0, 0)),
            pl.BlockSpec((NY * T_LOC, E_LOC), lambda e, f: (0, 0)),
            pl.BlockSpec((1, D, F_BLK), lambda e, f: (e, 0, f)),
            pl.BlockSpec((1, F_BLK, D), lambda e, f: (e, f, 0)),
        ],
        out_specs=pl.BlockSpec((NY * T_LOC, D), lambda e, f: (0, 0)),
        out_shape=jax.ShapeDtypeStruct((NY * T_LOC, D), jnp.float32),
    )(xall, wloc, W1, W2)


def _combine(partial):

    def body(p_ref, out_ref, comm_ref, send_sem, recv_sem):
        my_x = lax.axis_index("x")
        my_y = lax.axis_index("y")
        peer = (my_x, 1 - my_y)

        barrier = pltpu.get_barrier_semaphore()
        pl.semaphore_signal(barrier, inc=1, device_id=peer,
                            device_id_type=pl.DeviceIdType.MESH)
        pl.semaphore_wait(barrier, 1)

        rdma = pltpu.make_async_remote_copy(
            src_ref=p_ref.at[1 - my_y],
            dst_ref=comm_ref,
            send_sem=send_sem,
            recv_sem=recv_sem,
            device_id=peer,
            device_id_type=pl.DeviceIdType.MESH,
        )
        rdma.start()
        rdma.wait()
        out_ref[...] = p_ref[my_y] + comm_ref[...]

    return pl.pallas_call(
        body,
        out_shape=jax.ShapeDtypeStruct((T_LOC, D), jnp.float32),
        in_specs=[pl.BlockSpec(memory_space=pltpu.VMEM)],
        out_specs=pl.BlockSpec(memory_space=pltpu.VMEM),
        scratch_shapes=[
            pltpu.VMEM((T_LOC, D), jnp.float32),
            pltpu.SemaphoreType.DMA,
            pltpu.SemaphoreType.DMA,
        ],
        compiler_params=pltpu.CompilerParams(collective_id=1),
    )(partial)


def kernel(x, router, W1, W2):
    xall_slots, rall_slots = _exchange(x, router)
    xall = xall_slots.reshape(NY * T_LOC, D)
    rfull = jnp.concatenate([rall_slots[0], rall_slots[1]], axis=1)

    gates = jnp.dot(xall, rfull, precision=lax.Precision.HIGHEST)
    top_v, top_i = lax.top_k(gates, 2)
    w = jnp.exp(top_v - top_v.max(axis=1, keepdims=True))
    w = w / w.sum(axis=1, keepdims=True)

    my_y = lax.axis_index("y")
    eids = my_y * E_LOC + jnp.arange(E_LOC)
    wloc = jnp.sum(
        w[:, :, None] * (top_i[:, :, None] == eids[None, None, :]), axis=1
    )

    partial = _ffn(xall, wloc, W1, W2)
    return _combine(partial.reshape(NY, T_LOC, D))
